# baseline (speedup 1.0000x reference)
import math
import numpy as np
import jax
import jax.numpy as jnp
from jax.experimental import pallas as pl
from jax.experimental.pallas import tpu as pltpu

_G = math.exp(-1.0 / (2.0 * 1.5 * 1.5))
_K0 = _G / (1.0 + 2.0 * _G)
_K1 = 1.0 / (1.0 + 2.0 * _G)

_LANE = 128


def _bilinear_matrix(in_size: int, scale: int) -> np.ndarray:
    out_size = in_size * scale
    o = np.arange(out_size, dtype=np.float64)
    src = np.clip((o + 0.5) * (in_size / out_size) - 0.5, 0.0, None)
    i0 = np.minimum(np.floor(src).astype(np.int64), in_size - 1)
    i1 = np.minimum(i0 + 1, in_size - 1)
    wgt = src - i0
    m = np.zeros((out_size, in_size), dtype=np.float64)
    m[np.arange(out_size), i0] += 1.0 - wgt
    m[np.arange(out_size), i1] += wgt
    return m


def _blur_band_matrix(n: int) -> np.ndarray:
    g = np.zeros((n, n), dtype=np.float64)
    for i in range(n):
        for off, kk in ((-1, _K0), (0, _K1), (1, _K0)):
            j = i + off
            if j < 0:
                j = -j
            elif j > n - 1:
                j = 2 * (n - 1) - j
            g[i, j] += kk
    return g


def _block_diag(mat: np.ndarray, k: int) -> np.ndarray:
    r, c = mat.shape
    out = np.zeros((k * r, k * c), dtype=mat.dtype)
    for b in range(k):
        out[b * r:(b + 1) * r, b * c:(b + 1) * c] = mat
    return out


def _make_body(bch: int, pk: int, f: int, sh: int, sw: int):
    nq = bch // pk

    def _body(x_ref, r_ref, a_ref, o_ref):
        hq = x_ref.shape[1]
        wq = x_ref.shape[2]
        xb = x_ref[...].reshape(bch * hq, wq)
        t = jnp.dot(xb, r_ref[...], preferred_element_type=jnp.float32)
        t = t.astype(jnp.bfloat16).reshape(nq, pk * hq, f * sw)
        for q in range(nq):
            acc = jnp.dot(a_ref[0], t[q][:, 0:sw],
                          preferred_element_type=jnp.float32)
            for j in range(1, f):
                acc += jnp.dot(a_ref[j], t[q][:, j * sw:(j + 1) * sw],
                               preferred_element_type=jnp.float32)
            o_ref[q * pk:(q + 1) * pk] = acc.reshape(pk, sh, sw)

    return _body


def _blur_upsample_planes(xp: jax.Array, s: int) -> jax.Array:
    m, h, w = xp.shape
    sh, sw = s * h, s * w

    f = 1
    if w < _LANE and _LANE % w == 0 and h % (_LANE // w) == 0:
        f = _LANE // w
    hq, wq = h // f, f * w

    a_np = _bilinear_matrix(h, s) @ _blur_band_matrix(h)
    r_np = (_bilinear_matrix(w, s) @ _blur_band_matrix(w)).T

    pk = 1
    for cand in (8, 4, 2):
        if m % cand == 0 and cand * hq <= 256:
            pk = cand
            break

    r_bd = jnp.asarray(_block_diag(r_np, f), dtype=jnp.bfloat16)
    a_stack = np.stack([_block_diag(np.ascontiguousarray(a_np[:, j::f]), pk)
                        for j in range(f)])
    a_stack = jnp.asarray(a_stack, dtype=jnp.bfloat16)

    bch = pk
    for d in range(m, 0, -1):
        if m % d == 0 and d % pk == 0 and d * (h * w * 2 + sh * sw * 4) <= (16 << 20):
            if m // d >= 8 or d == m:
                bch = d
                break
    g = m // bch

    xq = xp.reshape(m, hq, wq)
    flops = m * (2 * sh * h * w + 2 * sh * w * sw)
    bytes_accessed = int(xq.size * 2 + m * sh * sw * 4 + a_stack.size * 2
                         + r_bd.size * 2)

    return pl.pallas_call(
        _make_body(bch, pk, f, sh, sw),
        out_shape=jax.ShapeDtypeStruct((m, sh, sw), jnp.float32),
        grid=(g,),
        in_specs=[
            pl.BlockSpec((bch, hq, wq), lambda i: (i, 0, 0)),
            pl.BlockSpec((wq, f * sw), lambda i: (0, 0),
                         pipeline_mode=pl.Buffered(1)),
            pl.BlockSpec((f, pk * sh, pk * hq), lambda i: (0, 0, 0),
                         pipeline_mode=pl.Buffered(1)),
        ],
        out_specs=pl.BlockSpec((bch, sh, sw), lambda i: (i, 0, 0)),
        compiler_params=pltpu.CompilerParams(
            dimension_semantics=("arbitrary",)),
        cost_estimate=pl.CostEstimate(flops=int(flops), transcendentals=0,
                                      bytes_accessed=bytes_accessed),
    )(xq, r_bd, a_stack)


def kernel(x):
    n, c, h, w = x.shape
    s = 2
    out = _blur_upsample_planes(x.reshape(n * c, h, w).astype(jnp.bfloat16), s)
    return out.reshape(n, c, s * h, s * w)


# speedup vs baseline: 1.5671x; 1.5671x over previous
import math
import numpy as np
import jax
import jax.numpy as jnp
from jax.experimental import pallas as pl
from jax.experimental.pallas import tpu as pltpu

_G = math.exp(-1.0 / (2.0 * 1.5 * 1.5))
_K0 = _G / (1.0 + 2.0 * _G)
_K1 = 1.0 / (1.0 + 2.0 * _G)


def _bilinear_matrix(in_size: int, scale: int) -> np.ndarray:
    out_size = in_size * scale
    o = np.arange(out_size, dtype=np.float64)
    src = np.clip((o + 0.5) * (in_size / out_size) - 0.5, 0.0, None)
    i0 = np.minimum(np.floor(src).astype(np.int64), in_size - 1)
    i1 = np.minimum(i0 + 1, in_size - 1)
    wgt = src - i0
    m = np.zeros((out_size, in_size), dtype=np.float64)
    m[np.arange(out_size), i0] += 1.0 - wgt
    m[np.arange(out_size), i1] += wgt
    return m


def _blur_band_matrix(n: int) -> np.ndarray:
    g = np.zeros((n, n), dtype=np.float64)
    for i in range(n):
        for off, kk in ((-1, _K0), (0, _K1), (1, _K0)):
            j = i + off
            if j < 0:
                j = -j
            elif j > n - 1:
                j = 2 * (n - 1) - j
            g[i, j] += kk
    return g


def _make_body(bch: int, pk: int, sh: int, sw: int):
    nq = bch // pk

    def _body(x_ref, r_ref, a_ref, o_ref):
        h = x_ref.shape[1]
        w = x_ref.shape[2]
        xb = x_ref[...].reshape(bch * h, w)
        t = jnp.dot(xb, r_ref[...], preferred_element_type=jnp.float32)
        t = t.astype(jnp.bfloat16).reshape(nq, pk * h, sw)
        a = a_ref[...]
        for q in range(nq):
            y = jnp.dot(a, t[q], preferred_element_type=jnp.float32)
            o_ref[q * pk:(q + 1) * pk] = y.reshape(pk, sh, sw)

    return _body


def _blur_upsample_planes(xp: jax.Array, s: int) -> jax.Array:
    m, h, w = xp.shape
    sh, sw = s * h, s * w

    a_np = _bilinear_matrix(h, s) @ _blur_band_matrix(h)
    r_np = (_bilinear_matrix(w, s) @ _blur_band_matrix(w)).T

    pk = 1
    for cand in (4, 2):
        if m % cand == 0 and cand * h <= 256:
            pk = cand
            break
    a_bd = np.zeros((pk * sh, pk * h), dtype=np.float64)
    for b in range(pk):
        a_bd[b * sh:(b + 1) * sh, b * h:(b + 1) * h] = a_np
    a_bd = jnp.asarray(a_bd, dtype=jnp.bfloat16)
    r_bf = jnp.asarray(r_np, dtype=jnp.bfloat16)

    bch = pk
    for d in range(m, 0, -1):
        if m % d == 0 and d % pk == 0 and d * (h * w * 2 + sh * sw * 4) <= (16 << 20):
            if m // d >= 8 or d == m:
                bch = d
                break
    g = m // bch

    flops = m * (2 * sh * h * w + 2 * sh * w * sw)
    bytes_accessed = int(xp.size * 2 + m * sh * sw * 4 + a_bd.size * 2
                         + r_bf.size * 2)

    return pl.pallas_call(
        _make_body(bch, pk, sh, sw),
        out_shape=jax.ShapeDtypeStruct((m, sh, sw), jnp.float32),
        grid=(g,),
        in_specs=[
            pl.BlockSpec((bch, h, w), lambda i: (i, 0, 0)),
            pl.BlockSpec((w, sw), lambda i: (0, 0),
                         pipeline_mode=pl.Buffered(1)),
            pl.BlockSpec((pk * sh, pk * h), lambda i: (0, 0),
                         pipeline_mode=pl.Buffered(1)),
        ],
        out_specs=pl.BlockSpec((bch, sh, sw), lambda i: (i, 0, 0)),
        compiler_params=pltpu.CompilerParams(
            dimension_semantics=("arbitrary",)),
        cost_estimate=pl.CostEstimate(flops=int(flops), transcendentals=0,
                                      bytes_accessed=bytes_accessed),
    )(xp, r_bf, a_bd)


def kernel(x):
    n, c, h, w = x.shape
    s = 2
    out = _blur_upsample_planes(x.reshape(n * c, h, w).astype(jnp.bfloat16), s)
    return out.reshape(n, c, s * h, s * w)
